# baseline (speedup 1.0000x reference)
import jax
import jax.numpy as jnp
from jax.experimental import pallas as pl


_BB = 8


def _body(inp_ref, tab_ref, out_ref):
    out_ref[...] = inp_ref[...] + tab_ref[...][None, :, :]


def kernel(inputs, table):
    B, _, D = inputs.shape
    S1 = table.shape[0]
    return pl.pallas_call(
        _body,
        grid=(B // _BB,),
        in_specs=[
            pl.BlockSpec((_BB, 1, D), lambda i: (i, 0, 0)),
            pl.BlockSpec((S1, D), lambda i: (0, 0)),
        ],
        out_specs=pl.BlockSpec((_BB, S1, D), lambda i: (i, 0, 0)),
        out_shape=jax.ShapeDtypeStruct((B, S1, D), jnp.float32),
    )(inputs, table)


# speedup vs baseline: 1.0057x; 1.0057x over previous
import jax
import jax.numpy as jnp
from jax.experimental import pallas as pl
from jax.experimental.pallas import tpu as pltpu


_BB = 8
_NBUF = 4


def _body(nsteps, inp_ref, tab_ref, out_ref, scratch, sems):
    i = pl.program_id(0)
    slot = jax.lax.rem(i, _NBUF)

    @pl.when(i >= _NBUF)
    def _wait_prev():
        j = i - _NBUF
        pltpu.make_async_copy(
            scratch.at[slot], out_ref.at[pl.ds(j * _BB, _BB)], sems.at[slot]
        ).wait()

    block = inp_ref[pl.ds(i * _BB, _BB)] + tab_ref[...][None, :, :]
    scratch[slot] = block
    pltpu.make_async_copy(
        scratch.at[slot], out_ref.at[pl.ds(i * _BB, _BB)], sems.at[slot]
    ).start()

    @pl.when(i == nsteps - 1)
    def _drain():
        for k in range(_NBUF):
            j = nsteps - _NBUF + k
            s = j % _NBUF
            pltpu.make_async_copy(
                scratch.at[s], out_ref.at[pl.ds(j * _BB, _BB)], sems.at[s]
            ).wait()


def kernel(inputs, table):
    B, _, D = inputs.shape
    S1 = table.shape[0]
    nsteps = B // _BB
    import functools
    return pl.pallas_call(
        functools.partial(_body, nsteps),
        grid=(nsteps,),
        in_specs=[
            pl.BlockSpec(memory_space=pltpu.VMEM),
            pl.BlockSpec(memory_space=pltpu.VMEM),
        ],
        out_specs=pl.BlockSpec(memory_space=pl.ANY),
        out_shape=jax.ShapeDtypeStruct((B, S1, D), jnp.float32),
        scratch_shapes=[
            pltpu.VMEM((_NBUF, _BB, S1, D), jnp.float32),
            pltpu.SemaphoreType.DMA((_NBUF,)),
        ],
    )(inputs, table)
